# baseline (speedup 1.0000x reference)
import functools
import math

import jax
import jax.numpy as jnp
from jax.experimental import pallas as pl
from jax.experimental.pallas import tpu as pltpu

L = 128


def _mlstm_chunk_kernel(H, Dh, NC,
                        x_ref, wq_ref, wk_ref, wv_ref, wi_ref, bi_ref,
                        wf_ref, bf_ref, wo_ref, bo_ref, wout_ref, g_ref,
                        be_ref, e_ref, bm_ref,
                        out_ref, c_out_ref, n_out_ref, c_s, n_s):
    c = pl.program_id(1)

    @pl.when(c == 0)
    def _():
        c_s[...] = jnp.zeros_like(c_s)
        n_s[...] = jnp.zeros_like(n_s)

    xb = x_ref[0]

    HI = jax.lax.Precision.HIGHEST

    def dot_t(a, b, prec=HI):
        return jax.lax.dot_general(a, b, (((1,), (1,)), ((), ())),
                                   preferred_element_type=jnp.float32,
                                   precision=prec)

    def dot_n(a, b, prec=HI):
        return jax.lax.dot_general(a, b, (((1,), (0,)), ((), ())),
                                   preferred_element_type=jnp.float32,
                                   precision=prec)

    q = dot_t(xb, wq_ref[...])
    k = dot_t(xb, wk_ref[...]) * (1.0 / math.sqrt(Dh))
    v = dot_t(xb, wv_ref[...])
    a_i = dot_t(xb, wi_ref[...]) + bi_ref[...]
    a_f = dot_t(xb, wf_ref[...]) + bf_ref[...]
    o = jax.nn.sigmoid(dot_t(xb, wo_ref[...]) + bo_ref[...])

    row = jax.lax.broadcasted_iota(jnp.int32, (L, L), 0)
    col = jax.lax.broadcasted_iota(jnp.int32, (L, L), 1)
    tril = col <= row
    tril_f = jnp.where(tril, 1.0, 0.0)
    lf = dot_n(tril_f, a_f)

    fv = jnp.exp(lf)
    wk_dec = jnp.exp(a_i - lf)

    ee = e_ref[...]
    qp = q * dot_n(fv, ee)
    kp = k * dot_n(wk_dec, ee)

    kcum = dot_n(tril_f, kp)
    n_prev = n_s[...]
    nq = dot_t(qp * (kcum + n_prev), ee)
    inv = 1.0 / jnp.maximum(jnp.abs(nq), 1.0)
    inv_rep = dot_n(inv, ee)

    cbd = c_s[...]
    h_inter = dot_t(qp, cbd)

    his = []
    for h in range(H):
        sl = slice(h * Dh, (h + 1) * Dh)
        s_mat = jnp.where(tril, dot_t(qp[:, sl], kp[:, sl]), 0.0)
        his.append(dot_n(s_mat, v[:, sl]))
    hi = jnp.concatenate(his, axis=1)

    hs = (hi + h_inter) * inv_rep * o

    f_last = dot_n(fv[L - 1:L, :], ee)
    m_full = jax.lax.dot_general(v, kp, (((0,), (0,)), ((), ())),
                                 preferred_element_type=jnp.float32,
                                 precision=HI)
    c_s[...] = f_last * (cbd + m_full * bm_ref[...])
    n_s[...] = f_last * (n_prev + jnp.sum(kp, axis=0, keepdims=True))

    mu = jnp.mean(hs, axis=-1, keepdims=True)
    var = jnp.mean((hs - mu) ** 2, axis=-1, keepdims=True)
    hn = (hs - mu) * jax.lax.rsqrt(var + 1e-5) * g_ref[...] + be_ref[...]
    out_ref[0] = dot_t(hn, wout_ref[...])

    @pl.when(c == NC - 1)
    def _():
        c_out_ref[0] = c_s[...]
        n_out_ref[0] = n_s[...]


def kernel(x, Wq, Wk, Wv, Wi, bi, Wf, bf, Wo, bo, W_out, ln_g, ln_b):
    B, T, IN = x.shape
    HD = Wq.shape[0]
    H = Wi.shape[0]
    Dh = HD // H
    HID = W_out.shape[0]
    NC = T // L
    f32 = jnp.float32

    lane = jnp.arange(HD, dtype=jnp.int32) // Dh
    ee = (lane[None, :] == jnp.arange(H, dtype=jnp.int32)[:, None]).astype(f32)
    bm = (lane[:, None] == lane[None, :]).astype(f32)

    body = functools.partial(_mlstm_chunk_kernel, H, Dh, NC)
    full = lambda shape: pl.BlockSpec(shape, lambda b, c: (0,) * len(shape))
    out, Cf, nf = pl.pallas_call(
        body,
        grid=(B, NC),
        in_specs=[
            pl.BlockSpec((1, L, IN), lambda b, c: (b, c, 0)),
            full((HD, IN)), full((HD, IN)), full((HD, IN)),
            full((H, IN)), full((1, H)),
            full((H, IN)), full((1, H)),
            full((HD, IN)), full((1, HD)),
            full((HID, HD)), full((1, HD)), full((1, HD)),
            full((H, HD)), full((HD, HD)),
        ],
        out_specs=[
            pl.BlockSpec((1, L, HID), lambda b, c: (b, c, 0)),
            pl.BlockSpec((1, HD, HD), lambda b, c: (b, 0, 0)),
            pl.BlockSpec((1, 1, HD), lambda b, c: (b, 0, 0)),
        ],
        out_shape=[
            jax.ShapeDtypeStruct((B, T, HID), f32),
            jax.ShapeDtypeStruct((B, HD, HD), f32),
            jax.ShapeDtypeStruct((B, 1, HD), f32),
        ],
        scratch_shapes=[
            pltpu.VMEM((HD, HD), f32),
            pltpu.VMEM((1, HD), f32),
        ],
        compiler_params=pltpu.CompilerParams(
            dimension_semantics=("parallel", "arbitrary"),
            vmem_limit_bytes=48 * 1024 * 1024,
        ),
        name="mlstm_chunk",
    )(x, Wq, Wk, Wv,
      Wi, bi.reshape(1, H), Wf, bf.reshape(1, H),
      Wo, bo.reshape(1, HD), W_out, ln_g.reshape(1, HD), ln_b.reshape(1, HD),
      ee, bm)

    idx = jnp.arange(H)
    C = Cf.reshape(B, H, Dh, H, Dh)[:, idx, :, idx, :].transpose(1, 0, 2, 3)
    n = nf.reshape(B, H, Dh)
    return out, (C, n)


# speedup vs baseline: 3.7597x; 3.7597x over previous
import functools
import math

import jax
import jax.numpy as jnp
from jax.experimental import pallas as pl
from jax.experimental.pallas import tpu as pltpu

L = 128


def _mlstm_chunk_kernel(H, Dh, NC,
                        x_ref, wq_ref, wk_ref, wv_ref, wi_ref, bi_ref,
                        wf_ref, bf_ref, wo_ref, bo_ref, wout_ref, g_ref,
                        be_ref, e_ref, bm_ref,
                        out_ref, c_out_ref, n_out_ref, c_s, n_s):
    c = pl.program_id(1)

    @pl.when(c == 0)
    def _():
        c_s[...] = jnp.zeros_like(c_s)
        n_s[...] = jnp.zeros_like(n_s)

    xb = x_ref[0]

    HI = jax.lax.Precision.HIGHEST
    DEF = jax.lax.Precision.DEFAULT

    def dot_t(a, b, prec=DEF):
        return jax.lax.dot_general(a, b, (((1,), (1,)), ((), ())),
                                   preferred_element_type=jnp.float32,
                                   precision=prec)

    def dot_n(a, b, prec=DEF):
        return jax.lax.dot_general(a, b, (((1,), (0,)), ((), ())),
                                   preferred_element_type=jnp.float32,
                                   precision=prec)

    q = dot_t(xb, wq_ref[...])
    k = dot_t(xb, wk_ref[...]) * (1.0 / math.sqrt(Dh))
    v = dot_t(xb, wv_ref[...])
    a_i = dot_t(xb, wi_ref[...]) + bi_ref[...]
    a_f = dot_t(xb, wf_ref[...]) + bf_ref[...]
    o = jax.nn.sigmoid(dot_t(xb, wo_ref[...]) + bo_ref[...])

    row = jax.lax.broadcasted_iota(jnp.int32, (L, L), 0)
    col = jax.lax.broadcasted_iota(jnp.int32, (L, L), 1)
    tril = col <= row
    tril_f = jnp.where(tril, 1.0, 0.0)
    lf = dot_n(tril_f, a_f, HI)

    fv = jnp.exp(lf)
    wk_dec = jnp.exp(a_i - lf)

    ee = e_ref[...]
    qp = q * dot_n(fv, ee, HI)
    kp = k * dot_n(wk_dec, ee, HI)

    n_prev = n_s[...]
    cbd = c_s[...]
    h_inter = dot_t(qp, cbd)

    his = []
    for h in range(H):
        sl = slice(h * Dh, (h + 1) * Dh)
        s_mat = jnp.where(tril, dot_t(qp[:, sl], kp[:, sl]), 0.0)
        nq = (jnp.sum(s_mat, axis=1, keepdims=True)
              + dot_t(qp[:, sl], n_prev[:, sl]))
        denom = jnp.maximum(jnp.abs(nq), 1.0)
        his.append((dot_n(s_mat, v[:, sl]) + h_inter[:, sl]) / denom)
    hs = jnp.concatenate(his, axis=1) * o

    f_last = dot_n(fv[L - 1:L, :], ee, HI)
    m_full = jax.lax.dot_general(v, kp, (((0,), (0,)), ((), ())),
                                 preferred_element_type=jnp.float32)
    c_s[...] = f_last * (cbd + m_full * bm_ref[...])
    n_s[...] = f_last * (n_prev + jnp.sum(kp, axis=0, keepdims=True))

    mu = jnp.mean(hs, axis=-1, keepdims=True)
    var = jnp.mean((hs - mu) ** 2, axis=-1, keepdims=True)
    hn = (hs - mu) * jax.lax.rsqrt(var + 1e-5) * g_ref[...] + be_ref[...]
    out_ref[0] = dot_t(hn, wout_ref[...])

    @pl.when(c == NC - 1)
    def _():
        c_out_ref[0] = c_s[...]
        n_out_ref[0] = n_s[...]


def kernel(x, Wq, Wk, Wv, Wi, bi, Wf, bf, Wo, bo, W_out, ln_g, ln_b):
    B, T, IN = x.shape
    HD = Wq.shape[0]
    H = Wi.shape[0]
    Dh = HD // H
    HID = W_out.shape[0]
    NC = T // L
    f32 = jnp.float32

    lane = jnp.arange(HD, dtype=jnp.int32) // Dh
    ee = (lane[None, :] == jnp.arange(H, dtype=jnp.int32)[:, None]).astype(f32)
    bm = (lane[:, None] == lane[None, :]).astype(f32)

    body = functools.partial(_mlstm_chunk_kernel, H, Dh, NC)
    full = lambda shape: pl.BlockSpec(shape, lambda b, c: (0,) * len(shape))
    out, Cf, nf = pl.pallas_call(
        body,
        grid=(B, NC),
        in_specs=[
            pl.BlockSpec((1, L, IN), lambda b, c: (b, c, 0)),
            full((HD, IN)), full((HD, IN)), full((HD, IN)),
            full((H, IN)), full((1, H)),
            full((H, IN)), full((1, H)),
            full((HD, IN)), full((1, HD)),
            full((HID, HD)), full((1, HD)), full((1, HD)),
            full((H, HD)), full((HD, HD)),
        ],
        out_specs=[
            pl.BlockSpec((1, L, HID), lambda b, c: (b, c, 0)),
            pl.BlockSpec((1, HD, HD), lambda b, c: (b, 0, 0)),
            pl.BlockSpec((1, 1, HD), lambda b, c: (b, 0, 0)),
        ],
        out_shape=[
            jax.ShapeDtypeStruct((B, T, HID), f32),
            jax.ShapeDtypeStruct((B, HD, HD), f32),
            jax.ShapeDtypeStruct((B, 1, HD), f32),
        ],
        scratch_shapes=[
            pltpu.VMEM((HD, HD), f32),
            pltpu.VMEM((1, HD), f32),
        ],
        compiler_params=pltpu.CompilerParams(
            dimension_semantics=("parallel", "arbitrary"),
            vmem_limit_bytes=48 * 1024 * 1024,
        ),
        name="mlstm_chunk",
    )(x, Wq, Wk, Wv,
      Wi, bi.reshape(1, H), Wf, bf.reshape(1, H),
      Wo, bo.reshape(1, HD), W_out, ln_g.reshape(1, HD), ln_b.reshape(1, HD),
      ee, bm)

    idx = jnp.arange(H)
    C = Cf.reshape(B, H, Dh, H, Dh)[:, idx, :, idx, :].transpose(1, 0, 2, 3)
    n = nf.reshape(B, H, Dh)
    return out, (C, n)


# speedup vs baseline: 3.8303x; 1.0188x over previous
import functools

import jax
import jax.numpy as jnp
from jax.experimental import pallas as pl
from jax.experimental.pallas import tpu as pltpu

L = 128


def _mlstm_chunk_kernel(H, Dh, NC,
                        x_ref, wq_ref, wk_ref, wv_ref, wi_ref, bi_ref,
                        wf_ref, bf_ref, wo_ref, bo_ref, wout_ref, g_ref,
                        be_ref, e_ref, bm_ref,
                        out_ref, c_out_ref, n_out_ref, c_s, n_s):
    c = pl.program_id(1)

    @pl.when(c == 0)
    def _():
        c_s[...] = jnp.zeros_like(c_s)
        n_s[...] = jnp.zeros_like(n_s)

    xb = x_ref[0]
    xb16 = xb.astype(jnp.bfloat16)

    HI = jax.lax.Precision.HIGHEST

    def dot_n(a, b, prec=None):
        return jax.lax.dot_general(a, b, (((1,), (0,)), ((), ())),
                                   preferred_element_type=jnp.float32,
                                   precision=prec)

    def dot_t(a, b, prec=None):
        return jax.lax.dot_general(a, b, (((1,), (1,)), ((), ())),
                                   preferred_element_type=jnp.float32,
                                   precision=prec)

    q = dot_n(xb16, wq_ref[...])
    k = dot_n(xb16, wk_ref[...])
    v = dot_n(xb16, wv_ref[...])
    a_i = dot_n(xb, wi_ref[...], HI) + bi_ref[...]
    a_f = dot_n(xb, wf_ref[...], HI) + bf_ref[...]
    o = jax.nn.sigmoid(dot_n(xb16, wo_ref[...]) + bo_ref[...])

    row = jax.lax.broadcasted_iota(jnp.int32, (L, L), 0)
    col = jax.lax.broadcasted_iota(jnp.int32, (L, L), 1)
    tril = col <= row
    tril_f = jnp.where(tril, 1.0, 0.0)
    lf = dot_n(tril_f, a_f, HI)

    fv = jnp.exp(lf)
    wk_dec = jnp.exp(a_i - lf)

    ee = e_ref[...]
    qp = q * dot_n(fv, ee, HI)
    kp = k * dot_n(wk_dec, ee, HI)

    n_prev = n_s[...]
    cbd = c_s[...]
    h_inter = dot_t(qp, cbd)

    his = []
    for h in range(H):
        sl = slice(h * Dh, (h + 1) * Dh)
        s_mat = jnp.where(tril, dot_t(qp[:, sl], kp[:, sl]), 0.0)
        nq = (jnp.sum(s_mat, axis=1, keepdims=True)
              + dot_t(qp[:, sl], n_prev[:, sl]))
        denom = jnp.maximum(jnp.abs(nq), 1.0)
        his.append((dot_n(s_mat, v[:, sl]) + h_inter[:, sl]) / denom)
    hs = jnp.concatenate(his, axis=1) * o

    f_last = dot_n(fv[L - 1:L, :], ee, HI)
    m_full = jax.lax.dot_general(v, kp, (((0,), (0,)), ((), ())),
                                 preferred_element_type=jnp.float32)
    c_s[...] = f_last * (cbd + m_full * bm_ref[...])
    n_s[...] = f_last * (n_prev + jnp.sum(kp, axis=0, keepdims=True))

    mu = jnp.mean(hs, axis=-1, keepdims=True)
    var = jnp.mean((hs - mu) ** 2, axis=-1, keepdims=True)
    hn = (hs - mu) * jax.lax.rsqrt(var + 1e-5) * g_ref[...] + be_ref[...]
    out_ref[0] = dot_n(hn.astype(jnp.bfloat16), wout_ref[...])

    @pl.when(c == NC - 1)
    def _():
        c_out_ref[0] = c_s[...]
        n_out_ref[0] = n_s[...]


def kernel(x, Wq, Wk, Wv, Wi, bi, Wf, bf, Wo, bo, W_out, ln_g, ln_b):
    B, T, IN = x.shape
    HD = Wq.shape[0]
    H = Wi.shape[0]
    Dh = HD // H
    HID = W_out.shape[0]
    NC = T // L
    f32 = jnp.float32
    bf16 = jnp.bfloat16

    lane = jnp.arange(HD, dtype=jnp.int32) // Dh
    ee = (lane[None, :] == jnp.arange(H, dtype=jnp.int32)[:, None]).astype(f32)
    bm = (lane[:, None] == lane[None, :]).astype(f32)

    wq_t = Wq.T.astype(bf16)
    wk_t = (Wk.T * (1.0 / jnp.sqrt(jnp.float32(Dh)))).astype(bf16)
    wv_t = Wv.T.astype(bf16)
    wo_t = Wo.T.astype(bf16)
    wout_t = W_out.T.astype(bf16)
    wi_t = Wi.T
    wf_t = Wf.T

    body = functools.partial(_mlstm_chunk_kernel, H, Dh, NC)
    full = lambda shape: pl.BlockSpec(shape, lambda b, c: (0,) * len(shape))
    out, Cf, nf = pl.pallas_call(
        body,
        grid=(B, NC),
        in_specs=[
            pl.BlockSpec((1, L, IN), lambda b, c: (b, c, 0)),
            full((IN, HD)), full((IN, HD)), full((IN, HD)),
            full((IN, H)), full((1, H)),
            full((IN, H)), full((1, H)),
            full((IN, HD)), full((1, HD)),
            full((HD, HID)), full((1, HD)), full((1, HD)),
            full((H, HD)), full((HD, HD)),
        ],
        out_specs=[
            pl.BlockSpec((1, L, HID), lambda b, c: (b, c, 0)),
            pl.BlockSpec((1, HD, HD), lambda b, c: (b, 0, 0)),
            pl.BlockSpec((1, 1, HD), lambda b, c: (b, 0, 0)),
        ],
        out_shape=[
            jax.ShapeDtypeStruct((B, T, HID), f32),
            jax.ShapeDtypeStruct((B, HD, HD), f32),
            jax.ShapeDtypeStruct((B, 1, HD), f32),
        ],
        scratch_shapes=[
            pltpu.VMEM((HD, HD), f32),
            pltpu.VMEM((1, HD), f32),
        ],
        compiler_params=pltpu.CompilerParams(
            dimension_semantics=("parallel", "arbitrary"),
            vmem_limit_bytes=48 * 1024 * 1024,
        ),
        name="mlstm_chunk",
    )(x, wq_t, wk_t, wv_t,
      wi_t, bi.reshape(1, H), wf_t, bf.reshape(1, H),
      wo_t, bo.reshape(1, HD), wout_t, ln_g.reshape(1, HD), ln_b.reshape(1, HD),
      ee, bm)

    idx = jnp.arange(H)
    C = Cf.reshape(B, H, Dh, H, Dh)[:, idx, :, idx, :].transpose(1, 0, 2, 3)
    n = nf.reshape(B, H, Dh)
    return out, (C, n)


# speedup vs baseline: 4.0890x; 1.0676x over previous
import functools

import jax
import jax.numpy as jnp
from jax.experimental import pallas as pl
from jax.experimental.pallas import tpu as pltpu

L = 128
BSUB = 4


def _mlstm_chunk_kernel(H, Dh, NC,
                        x_ref, wq_ref, wk_ref, wv_ref, wi_ref, bi_ref,
                        wf_ref, bf_ref, wo_ref, bo_ref, wout_ref, g_ref,
                        be_ref, e_ref, bm_ref, trilbig_ref,
                        out_ref, c_out_ref, n_out_ref, c_s, n_s):
    c = pl.program_id(1)
    HD = H * Dh
    M = BSUB * L

    @pl.when(c == 0)
    def _():
        c_s[...] = jnp.zeros_like(c_s)
        n_s[...] = jnp.zeros_like(n_s)

    xb = x_ref[...].reshape(M, x_ref.shape[2])
    xb16 = xb.astype(jnp.bfloat16)

    HI = jax.lax.Precision.HIGHEST

    def dot_n(a, b, prec=None):
        return jax.lax.dot_general(a, b, (((1,), (0,)), ((), ())),
                                   preferred_element_type=jnp.float32,
                                   precision=prec)

    def dot_t(a, b, prec=None):
        return jax.lax.dot_general(a, b, (((1,), (1,)), ((), ())),
                                   preferred_element_type=jnp.float32,
                                   precision=prec)

    q = dot_n(xb16, wq_ref[...])
    k = dot_n(xb16, wk_ref[...])
    v = dot_n(xb16, wv_ref[...])
    a_i = dot_n(xb, wi_ref[...], HI) + bi_ref[...]
    a_f = dot_n(xb, wf_ref[...], HI) + bf_ref[...]
    o = jax.nn.sigmoid(dot_n(xb16, wo_ref[...]) + bo_ref[...])

    lf = dot_n(trilbig_ref[...], a_f, HI)

    fv = jnp.exp(lf)
    wk_dec = jnp.exp(a_i - lf)

    ee = e_ref[...]
    qp = q * dot_n(fv, ee, HI)
    kp = k * dot_n(wk_dec, ee, HI)

    row = jax.lax.broadcasted_iota(jnp.int32, (L, L), 0)
    col = jax.lax.broadcasted_iota(jnp.int32, (L, L), 1)
    tril = col <= row

    his = []
    for bi in range(BSUB):
        rsl = slice(bi * L, (bi + 1) * L)
        qp_b, kp_b, v_b = qp[rsl], kp[rsl], v[rsl]
        n_prev = n_s[bi:bi + 1, :]
        cbd = c_s[bi]
        h_inter = dot_t(qp_b, cbd)
        for h in range(H):
            sl = slice(h * Dh, (h + 1) * Dh)
            s_mat = jnp.where(tril, dot_t(qp_b[:, sl], kp_b[:, sl]), 0.0)
            nq = (jnp.sum(s_mat, axis=1, keepdims=True)
                  + dot_t(qp_b[:, sl], n_prev[:, sl]))
            denom = jnp.maximum(jnp.abs(nq), 1.0)
            his.append((dot_n(s_mat, v_b[:, sl]) + h_inter[:, sl]) / denom)

        f_last = dot_n(fv[(bi + 1) * L - 1:(bi + 1) * L, :], ee, HI)
        m_full = jax.lax.dot_general(v_b, kp_b, (((0,), (0,)), ((), ())),
                                     preferred_element_type=jnp.float32)
        c_s[bi] = f_last * (cbd + m_full * bm_ref[...])
        n_s[bi:bi + 1, :] = f_last * (n_prev + jnp.sum(kp_b, axis=0,
                                                       keepdims=True))

    hs = jnp.concatenate(
        [jnp.concatenate(his[bi * H:(bi + 1) * H], axis=1)
         for bi in range(BSUB)], axis=0) * o

    mu = jnp.mean(hs, axis=-1, keepdims=True)
    var = jnp.mean((hs - mu) ** 2, axis=-1, keepdims=True)
    hn = (hs - mu) * jax.lax.rsqrt(var + 1e-5) * g_ref[...] + be_ref[...]
    res = dot_n(hn.astype(jnp.bfloat16), wout_ref[...])
    out_ref[...] = res.reshape(out_ref.shape)

    @pl.when(c == NC - 1)
    def _():
        c_out_ref[...] = c_s[...]
        n_out_ref[...] = n_s[...].reshape(n_out_ref.shape)


def kernel(x, Wq, Wk, Wv, Wi, bi, Wf, bf, Wo, bo, W_out, ln_g, ln_b):
    B, T, IN = x.shape
    HD = Wq.shape[0]
    H = Wi.shape[0]
    Dh = HD // H
    HID = W_out.shape[0]
    NC = T // L
    M = BSUB * L
    f32 = jnp.float32
    bf16 = jnp.bfloat16

    lane = jnp.arange(HD, dtype=jnp.int32) // Dh
    ee = (lane[None, :] == jnp.arange(H, dtype=jnp.int32)[:, None]).astype(f32)
    bm = (lane[:, None] == lane[None, :]).astype(f32)
    r = jnp.arange(M, dtype=jnp.int32)
    trilbig = ((r[:, None] // L == r[None, :] // L)
               & (r[None, :] <= r[:, None])).astype(f32)

    wq_t = Wq.T.astype(bf16)
    wk_t = (Wk.T * (1.0 / jnp.sqrt(jnp.float32(Dh)))).astype(bf16)
    wv_t = Wv.T.astype(bf16)
    wo_t = Wo.T.astype(bf16)
    wout_t = W_out.T.astype(bf16)
    wi_t = Wi.T
    wf_t = Wf.T

    body = functools.partial(_mlstm_chunk_kernel, H, Dh, NC)
    full = lambda shape: pl.BlockSpec(shape, lambda b, c: (0,) * len(shape))
    out, Cf, nf = pl.pallas_call(
        body,
        grid=(B // BSUB, NC),
        in_specs=[
            pl.BlockSpec((BSUB, L, IN), lambda b, c: (b, c, 0)),
            full((IN, HD)), full((IN, HD)), full((IN, HD)),
            full((IN, H)), full((1, H)),
            full((IN, H)), full((1, H)),
            full((IN, HD)), full((1, HD)),
            full((HD, HID)), full((1, HD)), full((1, HD)),
            full((H, HD)), full((HD, HD)), full((M, M)),
        ],
        out_specs=[
            pl.BlockSpec((BSUB, L, HID), lambda b, c: (b, c, 0)),
            pl.BlockSpec((BSUB, HD, HD), lambda b, c: (b, 0, 0)),
            pl.BlockSpec((BSUB, 1, HD), lambda b, c: (b, 0, 0)),
        ],
        out_shape=[
            jax.ShapeDtypeStruct((B, T, HID), f32),
            jax.ShapeDtypeStruct((B, HD, HD), f32),
            jax.ShapeDtypeStruct((B, 1, HD), f32),
        ],
        scratch_shapes=[
            pltpu.VMEM((BSUB, HD, HD), f32),
            pltpu.VMEM((BSUB, HD), f32),
        ],
        compiler_params=pltpu.CompilerParams(
            dimension_semantics=("parallel", "arbitrary"),
            vmem_limit_bytes=48 * 1024 * 1024,
        ),
        name="mlstm_chunk",
    )(x, wq_t, wk_t, wv_t,
      wi_t, bi.reshape(1, H), wf_t, bf.reshape(1, H),
      wo_t, bo.reshape(1, HD), wout_t, ln_g.reshape(1, HD), ln_b.reshape(1, HD),
      ee, bm, trilbig)

    idx = jnp.arange(H)
    C = Cf.reshape(B, H, Dh, H, Dh)[:, idx, :, idx, :].transpose(1, 0, 2, 3)
    n = nf.reshape(B, H, Dh)
    return out, (C, n)


# speedup vs baseline: 4.6229x; 1.1306x over previous
import functools

import jax
import jax.numpy as jnp
from jax.experimental import pallas as pl
from jax.experimental.pallas import tpu as pltpu

L = 128
BSUB = 4


def _mlstm_chunk_kernel(H, Dh, NC,
                        x_ref, wq_ref, wk_ref, wv_ref, wi_ref, bi_ref,
                        wo_ref, bo_ref, wout_ref, g_ref,
                        be_ref, e_ref, bm_ref, trilbig_ref,
                        out_ref, c_out_ref, n_out_ref, c_s, n_s):
    c = pl.program_id(1)
    HD = H * Dh
    M = BSUB * L

    @pl.when(c == 0)
    def _():
        c_s[...] = jnp.zeros_like(c_s)
        n_s[...] = jnp.zeros_like(n_s)

    xb = x_ref[...].reshape(M, x_ref.shape[2])
    xb16 = xb.astype(jnp.bfloat16)

    HI = jax.lax.Precision.HIGHEST

    def dot_n(a, b, prec=None):
        return jax.lax.dot_general(a, b, (((1,), (0,)), ((), ())),
                                   preferred_element_type=jnp.float32,
                                   precision=prec)

    def dot_t(a, b, prec=None):
        return jax.lax.dot_general(a, b, (((1,), (1,)), ((), ())),
                                   preferred_element_type=jnp.float32,
                                   precision=prec)

    MID = jax.lax.Precision.HIGHEST

    q = dot_n(xb16, wq_ref[...])
    k = dot_n(xb16, wk_ref[...])
    v = dot_n(xb16, wv_ref[...])
    a_if = dot_n(xb, wi_ref[...], MID) + bi_ref[...]
    a_i = a_if[:, :H]
    a_f = a_if[:, H:]
    o = jax.nn.sigmoid(dot_n(xb16, wo_ref[...]) + bo_ref[...])

    lf = dot_n(trilbig_ref[...], a_f, MID)

    fv = jnp.exp(lf)
    wk_dec = jnp.exp(a_i - lf)

    ee = e_ref[...]
    qp = q * dot_n(fv, ee)
    kp = k * dot_n(wk_dec, ee)

    row = jax.lax.broadcasted_iota(jnp.int32, (L, L), 0)
    col = jax.lax.broadcasted_iota(jnp.int32, (L, L), 1)
    tril = col <= row

    his = []
    for bi in range(BSUB):
        rsl = slice(bi * L, (bi + 1) * L)
        qp_b, kp_b, v_b = qp[rsl], kp[rsl], v[rsl]
        n_prev = n_s[bi:bi + 1, :]
        cbd = c_s[bi]
        h_inter = dot_t(qp_b, cbd)
        for h in range(H):
            sl = slice(h * Dh, (h + 1) * Dh)
            s_mat = jnp.where(tril, dot_t(qp_b[:, sl], kp_b[:, sl]), 0.0)
            nq = (jnp.sum(s_mat, axis=1, keepdims=True)
                  + dot_t(qp_b[:, sl], n_prev[:, sl]))
            denom = jnp.maximum(jnp.abs(nq), 1.0)
            his.append((dot_n(s_mat, v_b[:, sl]) + h_inter[:, sl]) / denom)

        f_last = dot_n(fv[(bi + 1) * L - 1:(bi + 1) * L, :], ee, HI)
        m_full = jax.lax.dot_general(v_b, kp_b, (((0,), (0,)), ((), ())),
                                     preferred_element_type=jnp.float32)
        c_s[bi] = f_last * (cbd + m_full * bm_ref[...])
        n_s[bi:bi + 1, :] = f_last * (n_prev + jnp.sum(kp_b, axis=0,
                                                       keepdims=True))

    hs = jnp.concatenate(
        [jnp.concatenate(his[bi * H:(bi + 1) * H], axis=1)
         for bi in range(BSUB)], axis=0) * o

    mu = jnp.mean(hs, axis=-1, keepdims=True)
    var = jnp.mean((hs - mu) ** 2, axis=-1, keepdims=True)
    hn = (hs - mu) * jax.lax.rsqrt(var + 1e-5) * g_ref[...] + be_ref[...]
    res = dot_n(hn.astype(jnp.bfloat16), wout_ref[...])
    out_ref[...] = res.reshape(out_ref.shape)

    @pl.when(c == NC - 1)
    def _():
        c_out_ref[...] = c_s[...]
        n_out_ref[...] = n_s[...].reshape(n_out_ref.shape)


def kernel(x, Wq, Wk, Wv, Wi, bi, Wf, bf, Wo, bo, W_out, ln_g, ln_b):
    B, T, IN = x.shape
    HD = Wq.shape[0]
    H = Wi.shape[0]
    Dh = HD // H
    HID = W_out.shape[0]
    NC = T // L
    M = BSUB * L
    f32 = jnp.float32
    bf16 = jnp.bfloat16

    lane = jnp.arange(HD, dtype=jnp.int32) // Dh
    ee = (lane[None, :] == jnp.arange(H, dtype=jnp.int32)[:, None]).astype(f32)
    bm = (lane[:, None] == lane[None, :]).astype(f32)
    r = jnp.arange(M, dtype=jnp.int32)
    trilbig = ((r[:, None] // L == r[None, :] // L)
               & (r[None, :] <= r[:, None])).astype(f32)

    wq_t = Wq.T.astype(bf16)
    wk_t = (Wk.T * (1.0 / jnp.sqrt(jnp.float32(Dh)))).astype(bf16)
    wv_t = Wv.T.astype(bf16)
    wo_t = Wo.T.astype(bf16)
    wout_t = W_out.T.astype(bf16)
    wif_t = jnp.concatenate([Wi.T, Wf.T], axis=1)
    bif = jnp.concatenate([bi, bf]).reshape(1, 2 * H)

    body = functools.partial(_mlstm_chunk_kernel, H, Dh, NC)
    full = lambda shape: pl.BlockSpec(shape, lambda b, c: (0,) * len(shape))
    out, Cf, nf = pl.pallas_call(
        body,
        grid=(B // BSUB, NC),
        in_specs=[
            pl.BlockSpec((BSUB, L, IN), lambda b, c: (b, c, 0)),
            full((IN, HD)), full((IN, HD)), full((IN, HD)),
            full((IN, 2 * H)), full((1, 2 * H)),
            full((IN, HD)), full((1, HD)),
            full((HD, HID)), full((1, HD)), full((1, HD)),
            full((H, HD)), full((HD, HD)), full((M, M)),
        ],
        out_specs=[
            pl.BlockSpec((BSUB, L, HID), lambda b, c: (b, c, 0)),
            pl.BlockSpec((BSUB, HD, HD), lambda b, c: (b, 0, 0)),
            pl.BlockSpec((BSUB, 1, HD), lambda b, c: (b, 0, 0)),
        ],
        out_shape=[
            jax.ShapeDtypeStruct((B, T, HID), f32),
            jax.ShapeDtypeStruct((B, HD, HD), f32),
            jax.ShapeDtypeStruct((B, 1, HD), f32),
        ],
        scratch_shapes=[
            pltpu.VMEM((BSUB, HD, HD), f32),
            pltpu.VMEM((BSUB, HD), f32),
        ],
        compiler_params=pltpu.CompilerParams(
            dimension_semantics=("parallel", "arbitrary"),
            vmem_limit_bytes=48 * 1024 * 1024,
        ),
        name="mlstm_chunk",
    )(x, wq_t, wk_t, wv_t,
      wif_t, bif,
      wo_t, bo.reshape(1, HD), wout_t, ln_g.reshape(1, HD), ln_b.reshape(1, HD),
      ee, bm, trilbig)

    idx = jnp.arange(H)
    C = Cf.reshape(B, H, Dh, H, Dh)[:, idx, :, idx, :].transpose(1, 0, 2, 3)
    n = nf.reshape(B, H, Dh)
    return out, (C, n)
